# baseline (speedup 1.0000x reference)
import functools

import jax
import jax.numpy as jnp
from jax import lax
from jax.experimental import pallas as pl
from jax.experimental.pallas import tpu as pltpu
from jax.experimental.pallas import tpu_sc as plsc

EMBED_DIM = 2048
NUM_CORES = 2
NUM_SUBCORES = 16
NUM_TILES = NUM_CORES * NUM_SUBCORES
ROWS_PER_CHUNK = 16


@functools.partial(jax.jit, static_argnames=("num_chunks",))
def _sc_embed(embed_weight, idx, num_chunks):
    rows_per_tile = num_chunks * ROWS_PER_CHUNK
    total_rows = NUM_TILES * rows_per_tile
    mesh = plsc.VectorSubcoreMesh(core_axis_name="c", subcore_axis_name="s")

    @functools.partial(
        pl.kernel,
        out_type=jax.ShapeDtypeStruct((total_rows, EMBED_DIM), jnp.float32),
        mesh=mesh,
        scratch_types=[
            pltpu.VMEM((num_chunks, ROWS_PER_CHUNK), jnp.int32),
            pltpu.VMEM((ROWS_PER_CHUNK, EMBED_DIM), jnp.float32),
            pltpu.VMEM((ROWS_PER_CHUNK, EMBED_DIM), jnp.float32),
            pltpu.SemaphoreType.DMA,
            pltpu.SemaphoreType.DMA,
            pltpu.SemaphoreType.DMA,
            pltpu.SemaphoreType.DMA,
        ],
    )
    def k(table_hbm, idx_hbm, out_hbm, idx_v, buf0, buf1, gs0, gs1, ss0, ss1):
        wid = lax.axis_index("s") * NUM_CORES + lax.axis_index("c")
        pltpu.sync_copy(idx_hbm.at[wid], idx_v)
        base = wid * rows_per_tile
        R = ROWS_PER_CHUNK

        def fire_gather(j, buf, sem):
            pltpu.async_copy(table_hbm.at[idx_v.at[j]], buf, sem)

        def wait_gather(j, buf, sem):
            pltpu.make_async_copy(table_hbm.at[idx_v.at[j]], buf, sem).wait()

        def fire_store(j, buf, sem):
            pltpu.async_copy(buf, out_hbm.at[pl.ds(base + j * R, R)], sem)

        def wait_store(j, buf, sem):
            pltpu.make_async_copy(
                buf, out_hbm.at[pl.ds(base + j * R, R)], sem
            ).wait()

        fire_gather(0, buf0, gs0)
        fire_gather(1, buf1, gs1)

        @pl.loop(0, num_chunks - 2, step=2)
        def _(j):
            wait_gather(j, buf0, gs0)
            fire_store(j, buf0, ss0)
            wait_gather(j + 1, buf1, gs1)
            fire_store(j + 1, buf1, ss1)
            wait_store(j, buf0, ss0)
            fire_gather(j + 2, buf0, gs0)
            wait_store(j + 1, buf1, ss1)
            fire_gather(j + 3, buf1, gs1)

        jl = num_chunks - 2
        wait_gather(jl, buf0, gs0)
        fire_store(jl, buf0, ss0)
        wait_gather(jl + 1, buf1, gs1)
        fire_store(jl + 1, buf1, ss1)
        wait_store(jl, buf0, ss0)
        wait_store(jl + 1, buf1, ss1)

    return k(embed_weight, idx)


def kernel(input_ids, embed_weight):
    batch, seq = input_ids.shape
    total = batch * seq
    num_chunks = total // (NUM_TILES * ROWS_PER_CHUNK)
    idx = input_ids.reshape(NUM_TILES, num_chunks, ROWS_PER_CHUNK)
    out = _sc_embed(embed_weight, idx, num_chunks)
    return out.reshape(batch, seq, embed_weight.shape[1])


# speedup vs baseline: 1.0168x; 1.0168x over previous
import functools

import jax
import jax.numpy as jnp
from jax import lax
from jax.experimental import pallas as pl
from jax.experimental.pallas import tpu as pltpu
from jax.experimental.pallas import tpu_sc as plsc

EMBED_DIM = 2048
NUM_CORES = 2
NUM_SUBCORES = 16
NUM_TILES = NUM_CORES * NUM_SUBCORES
ROWS_PER_CHUNK = 8
NBUF = 4


@functools.partial(jax.jit, static_argnames=("num_chunks",))
def _sc_embed(embed_weight, idx, num_chunks):
    rows_per_tile = num_chunks * ROWS_PER_CHUNK
    total_rows = NUM_TILES * rows_per_tile
    mesh = plsc.VectorSubcoreMesh(core_axis_name="c", subcore_axis_name="s")

    @functools.partial(
        pl.kernel,
        out_type=jax.ShapeDtypeStruct((total_rows, EMBED_DIM), jnp.float32),
        mesh=mesh,
        scratch_types=[
            pltpu.VMEM((num_chunks, ROWS_PER_CHUNK), jnp.int32),
        ]
        + [pltpu.VMEM((ROWS_PER_CHUNK, EMBED_DIM), jnp.float32)] * NBUF
        + [pltpu.SemaphoreType.DMA] * (2 * NBUF),
    )
    def k(table_hbm, idx_hbm, out_hbm, idx_v, *bufs_and_sems):
        bufs = bufs_and_sems[:NBUF]
        gsems = bufs_and_sems[NBUF : 2 * NBUF]
        ssems = bufs_and_sems[2 * NBUF :]
        wid = lax.axis_index("s") * NUM_CORES + lax.axis_index("c")
        pltpu.sync_copy(idx_hbm.at[wid], idx_v)
        base = wid * rows_per_tile
        R = ROWS_PER_CHUNK

        def fire_gather(j, i):
            pltpu.async_copy(table_hbm.at[idx_v.at[j]], bufs[i], gsems[i])

        def wait_gather(j, i):
            pltpu.make_async_copy(table_hbm.at[idx_v.at[j]], bufs[i], gsems[i]).wait()

        def fire_store(j, i):
            pltpu.async_copy(bufs[i], out_hbm.at[pl.ds(base + j * R, R)], ssems[i])

        def wait_store(j, i):
            pltpu.make_async_copy(
                bufs[i], out_hbm.at[pl.ds(base + j * R, R)], ssems[i]
            ).wait()

        for i in range(NBUF):
            fire_gather(i, i)

        @pl.loop(0, num_chunks - NBUF, step=NBUF)
        def _(j):
            for i in range(NBUF):
                wait_gather(j + i, i)
                fire_store(j + i, i)
            for i in range(NBUF):
                wait_store(j + i, i)
                fire_gather(j + NBUF + i, i)

        jl = num_chunks - NBUF
        for i in range(NBUF):
            wait_gather(jl + i, i)
            fire_store(jl + i, i)
        for i in range(NBUF):
            wait_store(jl + i, i)

    return k(embed_weight, idx)


def kernel(input_ids, embed_weight):
    batch, seq = input_ids.shape
    total = batch * seq
    num_chunks = total // (NUM_TILES * ROWS_PER_CHUNK)
    idx = input_ids.reshape(NUM_TILES, num_chunks, ROWS_PER_CHUNK)
    out = _sc_embed(embed_weight, idx, num_chunks)
    return out.reshape(batch, seq, embed_weight.shape[1])
